# baseline (speedup 1.0000x reference)
import functools

import jax
import jax.numpy as jnp
from jax import lax
from jax.experimental import pallas as pl
from jax.experimental.pallas import tpu as pltpu
from jax.experimental.pallas import tpu_sc as plsc

B = 16384
F = 26
V = 100000
ROWS = V + 2000
D = 50
DP = 64
H = 300

NC = 2
NS = 16
NW = NC * NS

TOTAL = B * F
GW = 128
GATHERS_PER_CHUNK = 4
CHUNK = GW * GATHERS_PER_CHUNK
PER_W = TOTAL // NW
CHUNKS_PER_W = PER_W // CHUNK
NBUF = 2


def _sc_gather(tables_pad, idx_flat):
    mesh = plsc.VectorSubcoreMesh(core_axis_name="c", subcore_axis_name="s")

    @functools.partial(
        pl.kernel,
        mesh=mesh,
        out_type=jax.ShapeDtypeStruct((TOTAL, DP), jnp.float32),
        compiler_params=pltpu.CompilerParams(use_tc_tiling_on_sc=False),
        scratch_types=[
            pltpu.VMEM((PER_W,), jnp.int32),
            pltpu.VMEM((NBUF, CHUNK, DP), jnp.float32),
            pltpu.SemaphoreType.DMA,
            pltpu.SemaphoreType.DMA,
        ],
    )
    def k(tab_hbm, idx_hbm, out_hbm, idx_v, rows_v, gsem, osem):
        wid = lax.axis_index("s") * NC + lax.axis_index("c")
        base = wid * PER_W
        pltpu.sync_copy(idx_hbm.at[pl.ds(base, PER_W)], idx_v)

        def fire(c, buf):
            for j in range(GATHERS_PER_CHUNK):
                pltpu.make_async_copy(
                    tab_hbm.at[idx_v.at[pl.ds(c * CHUNK + j * GW, GW)]],
                    rows_v.at[buf].at[pl.ds(j * GW, GW)],
                    gsem,
                ).start()

        def drain(c, buf):
            for j in range(GATHERS_PER_CHUNK):
                pltpu.make_async_copy(
                    tab_hbm.at[idx_v.at[pl.ds(c * CHUNK + j * GW, GW)]],
                    rows_v.at[buf].at[pl.ds(j * GW, GW)],
                    gsem,
                ).wait()

        def out_copy(c, buf):
            return pltpu.make_async_copy(
                rows_v.at[buf],
                out_hbm.at[pl.ds(base + c * CHUNK, CHUNK)],
                osem,
            )

        fire(0, 0)

        @pl.loop(0, CHUNKS_PER_W, step=NBUF)
        def _(cbase):
            for bstat in range(NBUF):
                c = cbase + bstat
                nb = (bstat + 1) % NBUF

                @pl.when(c + 1 < CHUNKS_PER_W)
                def _(c=c, nb=nb):
                    @pl.when(c + 1 >= NBUF)
                    def _():
                        out_copy(c + 1 - NBUF, nb).wait()

                    fire(c + 1, nb)

                drain(c, bstat)
                out_copy(c, bstat).start()

        for t in range(NBUF):
            c = CHUNKS_PER_W - NBUF + t
            out_copy(c, c % NBUF).wait()

    return k(tables_pad, idx_flat)


def _pad_block(x_ref, o_ref):
    x = x_ref[...]
    o_ref[...] = jnp.concatenate(
        [x, jnp.zeros((x.shape[0], DP - D), x.dtype)], axis=1)


def _tc_pad_tables(tables_flat):
    BLKR = 2000
    n = F * ROWS
    return pl.pallas_call(
        _pad_block,
        grid=(n // BLKR,),
        in_specs=[pl.BlockSpec((BLKR, D), lambda i: (i, 0))],
        out_specs=pl.BlockSpec((BLKR, DP), lambda i: (i, 0)),
        out_shape=jax.ShapeDtypeStruct((n, DP), jnp.float32),
    )(tables_flat)


def _mlp_block(x_ref, w1_ref, b1_ref, w2_ref, b2_ref, o_ref):
    x = x_ref[...].astype(jnp.bfloat16)
    w1 = w1_ref[...].astype(jnp.bfloat16)
    l1 = jnp.dot(x, w1, preferred_element_type=jnp.float32) + b1_ref[...]
    l1 = jnp.maximum(l1, 0.0).astype(jnp.bfloat16)
    w2 = w2_ref[...].astype(jnp.bfloat16)
    l2 = jnp.dot(l1, w2, preferred_element_type=jnp.float32) + b2_ref[...]
    o_ref[...] = jax.nn.sigmoid(l2)


def _tc_mlp(x, W1p, b1, W2, b2):
    BLK = 1024
    return pl.pallas_call(
        _mlp_block,
        grid=(B // BLK,),
        in_specs=[
            pl.BlockSpec((BLK, F * DP), lambda i: (i, 0)),
            pl.BlockSpec((F * DP, H), lambda i: (0, 0)),
            pl.BlockSpec((1, H), lambda i: (0, 0)),
            pl.BlockSpec((H, 1), lambda i: (0, 0)),
            pl.BlockSpec((1, 1), lambda i: (0, 0)),
        ],
        out_specs=pl.BlockSpec((BLK, 1), lambda i: (i, 0)),
        out_shape=jax.ShapeDtypeStruct((B, 1), jnp.float32),
    )(x, W1p, b1, W2, b2)


def kernel(xb, tables, W1, b1, W2, b2):
    tables_pad = _tc_pad_tables(tables.reshape(F * ROWS, D))
    W1p = jnp.pad(W1.reshape(F, D, H), ((0, 0), (0, DP - D), (0, 0))).reshape(
        F * DP, H)
    idx_flat = (xb + (jnp.arange(F, dtype=jnp.int32) * ROWS)[None, :]).reshape(
        TOTAL)
    gathered = _sc_gather(tables_pad, idx_flat)
    x = gathered.reshape(B, F * DP)
    return _tc_mlp(x, W1p, b1.reshape(1, H), W2, b2.reshape(1, 1))


# speedup vs baseline: 1.2042x; 1.2042x over previous
import functools

import jax
import jax.numpy as jnp
from jax import lax
from jax.experimental import pallas as pl
from jax.experimental.pallas import tpu as pltpu
from jax.experimental.pallas import tpu_sc as plsc

B = 16384
F = 26
V = 100000
ROWS = V + 2000
D = 50
DP = 128
H = 300

NC = 2
NS = 16
NW = NC * NS

TOTAL = B * F
GW = 128
GATHERS_PER_CHUNK = 2
CHUNK = GW * GATHERS_PER_CHUNK
PER_W = TOTAL // NW
CHUNKS_PER_W = PER_W // CHUNK
NBUF = 2


def _sc_gather(tables_pad, idx_flat):
    mesh = plsc.VectorSubcoreMesh(core_axis_name="c", subcore_axis_name="s")

    @functools.partial(
        pl.kernel,
        mesh=mesh,
        out_type=jax.ShapeDtypeStruct((TOTAL, DP), jnp.float32),
        compiler_params=pltpu.CompilerParams(use_tc_tiling_on_sc=False),
        scratch_types=[
            pltpu.VMEM((PER_W,), jnp.int32),
            pltpu.VMEM((NBUF, CHUNK, DP), jnp.float32),
            pltpu.SemaphoreType.DMA,
            pltpu.SemaphoreType.DMA,
        ],
    )
    def k(tab_hbm, idx_hbm, out_hbm, idx_v, rows_v, gsem, osem):
        wid = lax.axis_index("s") * NC + lax.axis_index("c")
        base = wid * PER_W
        pltpu.sync_copy(idx_hbm.at[pl.ds(base, PER_W)], idx_v)

        def fire(c, buf):
            for j in range(GATHERS_PER_CHUNK):
                pltpu.make_async_copy(
                    tab_hbm.at[idx_v.at[pl.ds(c * CHUNK + j * GW, GW)]],
                    rows_v.at[buf].at[pl.ds(j * GW, GW)],
                    gsem,
                ).start()

        def drain(c, buf):
            for j in range(GATHERS_PER_CHUNK):
                pltpu.make_async_copy(
                    tab_hbm.at[idx_v.at[pl.ds(c * CHUNK + j * GW, GW)]],
                    rows_v.at[buf].at[pl.ds(j * GW, GW)],
                    gsem,
                ).wait()

        def out_copy(c, buf):
            return pltpu.make_async_copy(
                rows_v.at[buf],
                out_hbm.at[pl.ds(base + c * CHUNK, CHUNK)],
                osem,
            )

        fire(0, 0)

        @pl.loop(0, CHUNKS_PER_W, step=NBUF)
        def _(cbase):
            for bstat in range(NBUF):
                c = cbase + bstat
                nb = (bstat + 1) % NBUF

                @pl.when(c + 1 < CHUNKS_PER_W)
                def _(c=c, nb=nb):
                    @pl.when(c + 1 >= NBUF)
                    def _():
                        out_copy(c + 1 - NBUF, nb).wait()

                    fire(c + 1, nb)

                drain(c, bstat)
                out_copy(c, bstat).start()

        for t in range(NBUF):
            c = CHUNKS_PER_W - NBUF + t
            out_copy(c, c % NBUF).wait()

    return k(tables_pad, idx_flat)


def _pad_block(x_ref, o_ref):
    x = x_ref[...]
    o_ref[...] = jnp.concatenate(
        [x, jnp.zeros((x.shape[0], DP - D), x.dtype)], axis=1)


def _tc_pad_tables(tables_flat):
    BLKR = 2000
    n = F * ROWS
    return pl.pallas_call(
        _pad_block,
        grid=(n // BLKR,),
        in_specs=[pl.BlockSpec((BLKR, D), lambda i: (i, 0))],
        out_specs=pl.BlockSpec((BLKR, DP), lambda i: (i, 0)),
        out_shape=jax.ShapeDtypeStruct((n, DP), jnp.float32),
    )(tables_flat)


def _mlp_block(x_ref, w1_ref, b1_ref, w2_ref, b2_ref, o_ref):
    x = x_ref[...].astype(jnp.bfloat16)
    w1 = w1_ref[...].astype(jnp.bfloat16)
    l1 = jnp.dot(x, w1, preferred_element_type=jnp.float32) + b1_ref[...]
    l1 = jnp.maximum(l1, 0.0).astype(jnp.bfloat16)
    w2 = w2_ref[...].astype(jnp.bfloat16)
    l2 = jnp.dot(l1, w2, preferred_element_type=jnp.float32) + b2_ref[...]
    o_ref[...] = jax.nn.sigmoid(l2)


def _tc_mlp(x, W1p, b1, W2, b2):
    BLK = 1024
    return pl.pallas_call(
        _mlp_block,
        grid=(B // BLK,),
        in_specs=[
            pl.BlockSpec((BLK, F * DP), lambda i: (i, 0)),
            pl.BlockSpec((F * DP, H), lambda i: (0, 0)),
            pl.BlockSpec((1, H), lambda i: (0, 0)),
            pl.BlockSpec((H, 1), lambda i: (0, 0)),
            pl.BlockSpec((1, 1), lambda i: (0, 0)),
        ],
        out_specs=pl.BlockSpec((BLK, 1), lambda i: (i, 0)),
        out_shape=jax.ShapeDtypeStruct((B, 1), jnp.float32),
    )(x, W1p, b1, W2, b2)


def kernel(xb, tables, W1, b1, W2, b2):
    tables_pad = _tc_pad_tables(tables.reshape(F * ROWS, D))
    W1p = jnp.pad(W1.reshape(F, D, H), ((0, 0), (0, DP - D), (0, 0))).reshape(
        F * DP, H)
    idx_flat = (xb + (jnp.arange(F, dtype=jnp.int32) * ROWS)[None, :]).reshape(
        TOTAL)
    gathered = _sc_gather(tables_pad, idx_flat)
    x = gathered.reshape(B, F * DP)
    return _tc_mlp(x, W1p, b1.reshape(1, H), W2, b2.reshape(1, 1))


# speedup vs baseline: 2.8000x; 2.3252x over previous
import functools

import jax
import jax.numpy as jnp
from jax import lax
from jax.experimental import pallas as pl
from jax.experimental.pallas import tpu as pltpu
from jax.experimental.pallas import tpu_sc as plsc

B = 16384
F = 26
V = 100000
ROWS = V + 2000
D = 50
DP = 128
H = 300

NC = 2
NS = 16
NW = NC * NS

TOTAL = B * F
GW = 128
GATHERS_PER_CHUNK = 2
CHUNK = GW * GATHERS_PER_CHUNK
PER_W = TOTAL // NW
CHUNKS_PER_W = PER_W // CHUNK
NBUF = 2


def _sc_gather(tables_pad, idx_flat):
    mesh = plsc.VectorSubcoreMesh(core_axis_name="c", subcore_axis_name="s")

    @functools.partial(
        pl.kernel,
        mesh=mesh,
        out_type=jax.ShapeDtypeStruct((TOTAL, DP), jnp.float32),
        compiler_params=pltpu.CompilerParams(use_tc_tiling_on_sc=False),
        scratch_types=[
            pltpu.VMEM((PER_W,), jnp.int32),
            pltpu.VMEM((NBUF, CHUNK, DP), jnp.float32),
            pltpu.SemaphoreType.DMA,
            pltpu.SemaphoreType.DMA,
        ],
    )
    def k(tab_hbm, idx_hbm, out_hbm, idx_v, rows_v, gsem, osem):
        wid = lax.axis_index("s") * NC + lax.axis_index("c")
        base = wid * PER_W
        pltpu.sync_copy(idx_hbm.at[pl.ds(base, PER_W)], idx_v)

        def fire(c, buf):
            for j in range(GATHERS_PER_CHUNK):
                pltpu.make_async_copy(
                    tab_hbm.at[idx_v.at[pl.ds(c * CHUNK + j * GW, GW)]],
                    rows_v.at[buf].at[pl.ds(j * GW, GW)],
                    gsem,
                ).start()

        def drain(c, buf):
            for j in range(GATHERS_PER_CHUNK):
                pltpu.make_async_copy(
                    tab_hbm.at[idx_v.at[pl.ds(c * CHUNK + j * GW, GW)]],
                    rows_v.at[buf].at[pl.ds(j * GW, GW)],
                    gsem,
                ).wait()

        def out_copy(c, buf):
            return pltpu.make_async_copy(
                rows_v.at[buf],
                out_hbm.at[pl.ds(base + c * CHUNK, CHUNK)],
                osem,
            )

        fire(0, 0)

        @pl.loop(0, CHUNKS_PER_W, step=NBUF)
        def _(cbase):
            for bstat in range(NBUF):
                c = cbase + bstat
                nb = (bstat + 1) % NBUF

                @pl.when(c + 1 < CHUNKS_PER_W)
                def _(c=c, nb=nb):
                    @pl.when(c + 1 >= NBUF)
                    def _():
                        out_copy(c + 1 - NBUF, nb).wait()

                    fire(c + 1, nb)

                drain(c, bstat)
                out_copy(c, bstat).start()

        for t in range(NBUF):
            c = CHUNKS_PER_W - NBUF + t
            out_copy(c, c % NBUF).wait()

    return k(tables_pad, idx_flat)


_BLKC = 2048
ROWS2 = 102400


def _tpad_block(x_ref, o_ref):
    xt = x_ref[0].T
    o_ref[0] = jnp.concatenate(
        [xt, jnp.zeros((_BLKC, DP - D), xt.dtype)], axis=1)


def _tc_transpose_pad(tables_t):
    nj = ROWS2 // _BLKC
    return pl.pallas_call(
        _tpad_block,
        grid=(F, nj),
        in_specs=[pl.BlockSpec((1, D, _BLKC), lambda f, j: (f, 0, j))],
        out_specs=pl.BlockSpec((1, _BLKC, DP), lambda f, j: (f, j, 0)),
        out_shape=jax.ShapeDtypeStruct((F, ROWS2, DP), jnp.float32),
    )(tables_t)


def _mlp_block(x_ref, w1_ref, b1_ref, w2_ref, b2_ref, o_ref):
    x = x_ref[...].astype(jnp.bfloat16)
    w1 = w1_ref[...].astype(jnp.bfloat16)
    l1 = jnp.dot(x, w1, preferred_element_type=jnp.float32) + b1_ref[...]
    l1 = jnp.maximum(l1, 0.0).astype(jnp.bfloat16)
    w2 = w2_ref[...].astype(jnp.bfloat16)
    l2 = jnp.dot(l1, w2, preferred_element_type=jnp.float32) + b2_ref[...]
    o_ref[...] = jax.nn.sigmoid(l2)


def _tc_mlp(x, W1p, b1, W2, b2):
    BLK = 1024
    return pl.pallas_call(
        _mlp_block,
        grid=(B // BLK,),
        in_specs=[
            pl.BlockSpec((BLK, F * DP), lambda i: (i, 0)),
            pl.BlockSpec((F * DP, H), lambda i: (0, 0)),
            pl.BlockSpec((1, H), lambda i: (0, 0)),
            pl.BlockSpec((H, 1), lambda i: (0, 0)),
            pl.BlockSpec((1, 1), lambda i: (0, 0)),
        ],
        out_specs=pl.BlockSpec((BLK, 1), lambda i: (i, 0)),
        out_shape=jax.ShapeDtypeStruct((B, 1), jnp.float32),
    )(x, W1p, b1, W2, b2)


def kernel(xb, tables, W1, b1, W2, b2):
    tables_pad = _tc_transpose_pad(jnp.transpose(tables, (0, 2, 1))).reshape(
        F * ROWS2, DP)
    W1p = jnp.pad(W1.reshape(F, D, H), ((0, 0), (0, DP - D), (0, 0))).reshape(
        F * DP, H)
    idx_flat = (xb + (jnp.arange(F, dtype=jnp.int32) * ROWS2)[None, :]).reshape(
        TOTAL)
    gathered = _sc_gather(tables_pad, idx_flat)
    x = gathered.reshape(B, F * DP)
    return _tc_mlp(x, W1p, b1.reshape(1, H), W2, b2.reshape(1, 1))


# speedup vs baseline: 3.5036x; 1.2513x over previous
import functools

import jax
import jax.numpy as jnp
from jax import lax
from jax.experimental import pallas as pl
from jax.experimental.pallas import tpu as pltpu
from jax.experimental.pallas import tpu_sc as plsc

B = 16384
F = 26
V = 100000
ROWS = V + 2000
D = 50
DP = 128
H = 300

NC = 2
NS = 16
NW = NC * NS

TOTAL = B * F
GW = 128
GATHERS_PER_CHUNK = 2
CHUNK = GW * GATHERS_PER_CHUNK
PER_W = TOTAL // NW
CHUNKS_PER_W = PER_W // CHUNK
NBUF = 2


def _sc_gather(tables_pad, idx_flat):
    mesh = plsc.VectorSubcoreMesh(core_axis_name="c", subcore_axis_name="s")

    @functools.partial(
        pl.kernel,
        mesh=mesh,
        out_type=jax.ShapeDtypeStruct((TOTAL, DP), jnp.float32),
        compiler_params=pltpu.CompilerParams(use_tc_tiling_on_sc=False),
        scratch_types=[
            pltpu.VMEM((PER_W,), jnp.int32),
            pltpu.VMEM((NBUF, CHUNK, DP), jnp.float32),
            pltpu.SemaphoreType.DMA,
            pltpu.SemaphoreType.DMA,
        ],
    )
    def k(tab_hbm, idx_hbm, out_hbm, idx_v, rows_v, gsem, osem):
        wid = lax.axis_index("s") * NC + lax.axis_index("c")
        base = wid * PER_W
        pltpu.sync_copy(idx_hbm.at[pl.ds(base, PER_W)], idx_v)

        def fire(c, buf):
            for j in range(GATHERS_PER_CHUNK):
                pltpu.make_async_copy(
                    tab_hbm.at[idx_v.at[pl.ds(c * CHUNK + j * GW, GW)]],
                    rows_v.at[buf].at[pl.ds(j * GW, GW)],
                    gsem,
                ).start()

        def drain(c, buf):
            for j in range(GATHERS_PER_CHUNK):
                pltpu.make_async_copy(
                    tab_hbm.at[idx_v.at[pl.ds(c * CHUNK + j * GW, GW)]],
                    rows_v.at[buf].at[pl.ds(j * GW, GW)],
                    gsem,
                ).wait()

        def out_copy(c, buf):
            return pltpu.make_async_copy(
                rows_v.at[buf],
                out_hbm.at[pl.ds(base + c * CHUNK, CHUNK)],
                osem,
            )

        fire(0, 0)

        @pl.loop(0, CHUNKS_PER_W, step=NBUF)
        def _(cbase):
            for bstat in range(NBUF):
                c = cbase + bstat
                nb = (bstat + 1) % NBUF

                @pl.when(c + 1 < CHUNKS_PER_W)
                def _(c=c, nb=nb):
                    @pl.when(c + 1 >= NBUF)
                    def _():
                        out_copy(c + 1 - NBUF, nb).wait()

                    fire(c + 1, nb)

                drain(c, bstat)
                out_copy(c, bstat).start()

        for t in range(NBUF):
            c = CHUNKS_PER_W - NBUF + t
            out_copy(c, c % NBUF).wait()

    return k(tables_pad, idx_flat)


_BLKC = 4096
ROWS2 = 102400


def _tpad_block(x_ref, o_ref):
    xt = x_ref[0].T
    o_ref[0] = jnp.concatenate(
        [xt, jnp.zeros((_BLKC, DP - D), xt.dtype)], axis=1)


def _tc_transpose_pad(tables_t):
    nj = ROWS2 // _BLKC
    return pl.pallas_call(
        _tpad_block,
        grid=(F, nj),
        in_specs=[pl.BlockSpec((1, D, _BLKC), lambda f, j: (f, 0, j))],
        out_specs=pl.BlockSpec((1, _BLKC, DP), lambda f, j: (f, j, 0)),
        out_shape=jax.ShapeDtypeStruct((F, ROWS2, DP), jnp.float32),
    )(tables_t)


def _mlp_block(x_ref, w1_ref, b1_ref, w2_ref, b2_ref, o_ref):
    x = x_ref[...].astype(jnp.bfloat16)
    w1 = w1_ref[...].astype(jnp.bfloat16)
    l1 = jnp.dot(x, w1, preferred_element_type=jnp.float32) + b1_ref[...]
    l1 = jnp.maximum(l1, 0.0).astype(jnp.bfloat16)
    w2 = w2_ref[...].astype(jnp.bfloat16)
    l2 = jnp.dot(l1, w2, preferred_element_type=jnp.float32) + b2_ref[...]
    o_ref[...] = jax.nn.sigmoid(l2)


def _tc_mlp(x, W1p, b1, W2, b2):
    BLK = 1024
    return pl.pallas_call(
        _mlp_block,
        grid=(B // BLK,),
        in_specs=[
            pl.BlockSpec((BLK, F * DP), lambda i: (i, 0)),
            pl.BlockSpec((F * DP, H), lambda i: (0, 0)),
            pl.BlockSpec((1, H), lambda i: (0, 0)),
            pl.BlockSpec((H, 1), lambda i: (0, 0)),
            pl.BlockSpec((1, 1), lambda i: (0, 0)),
        ],
        out_specs=pl.BlockSpec((BLK, 1), lambda i: (i, 0)),
        out_shape=jax.ShapeDtypeStruct((B, 1), jnp.float32),
    )(x, W1p, b1, W2, b2)


def kernel(xb, tables, W1, b1, W2, b2):
    tables_pad = _tc_transpose_pad(jnp.transpose(tables, (0, 2, 1))).reshape(
        F * ROWS2, DP)
    W1p = jnp.pad(W1.reshape(F, D, H), ((0, 0), (0, DP - D), (0, 0))).reshape(
        F * DP, H)
    idx_flat = (xb + (jnp.arange(F, dtype=jnp.int32) * ROWS2)[None, :]).reshape(
        TOTAL)
    gathered = _sc_gather(tables_pad, idx_flat)
    x = gathered.reshape(B, F * DP)
    return _tc_mlp(x, W1p, b1.reshape(1, H), W2, b2.reshape(1, 1))


# speedup vs baseline: 4.1939x; 1.1970x over previous
import functools

import jax
import jax.numpy as jnp
from jax import lax
from jax.experimental import pallas as pl
from jax.experimental.pallas import tpu as pltpu
from jax.experimental.pallas import tpu_sc as plsc

B = 16384
F = 26
V = 100000
ROWS = V + 2000
D = 50
DP = 128
H = 300

NC = 2
NS = 16
NW = NC * NS

TOTAL = B * F
GW = 128
GATHERS_PER_CHUNK = 2
CHUNK = GW * GATHERS_PER_CHUNK
PER_W = TOTAL // NW
CHUNKS_PER_W = PER_W // CHUNK
NBUF = 2


def _sc_gather(tables_pad, idx_flat):
    mesh = plsc.VectorSubcoreMesh(core_axis_name="c", subcore_axis_name="s")

    @functools.partial(
        pl.kernel,
        mesh=mesh,
        out_type=jax.ShapeDtypeStruct((TOTAL, DP), jnp.float32),
        compiler_params=pltpu.CompilerParams(use_tc_tiling_on_sc=False),
        scratch_types=[
            pltpu.VMEM((PER_W,), jnp.int32),
            pltpu.VMEM((NBUF, CHUNK, DP), jnp.float32),
            pltpu.SemaphoreType.DMA,
            pltpu.SemaphoreType.DMA,
        ],
    )
    def k(tab_hbm, idx_hbm, out_hbm, idx_v, rows_v, gsem, osem):
        wid = lax.axis_index("s") * NC + lax.axis_index("c")
        base = wid * PER_W
        pltpu.sync_copy(idx_hbm.at[pl.ds(base, PER_W)], idx_v)

        def fire(c, buf):
            for j in range(GATHERS_PER_CHUNK):
                pltpu.make_async_copy(
                    tab_hbm.at[idx_v.at[pl.ds(c * CHUNK + j * GW, GW)]],
                    rows_v.at[buf].at[pl.ds(j * GW, GW)],
                    gsem,
                ).start()

        def drain(c, buf):
            for j in range(GATHERS_PER_CHUNK):
                pltpu.make_async_copy(
                    tab_hbm.at[idx_v.at[pl.ds(c * CHUNK + j * GW, GW)]],
                    rows_v.at[buf].at[pl.ds(j * GW, GW)],
                    gsem,
                ).wait()

        def out_copy(c, buf):
            return pltpu.make_async_copy(
                rows_v.at[buf],
                out_hbm.at[pl.ds(base + c * CHUNK, CHUNK)],
                osem,
            )

        fire(0, 0)

        @pl.loop(0, CHUNKS_PER_W, step=NBUF)
        def _(cbase):
            for bstat in range(NBUF):
                c = cbase + bstat
                nb = (bstat + 1) % NBUF

                @pl.when(c + 1 < CHUNKS_PER_W)
                def _(c=c, nb=nb):
                    @pl.when(c + 1 >= NBUF)
                    def _():
                        out_copy(c + 1 - NBUF, nb).wait()

                    fire(c + 1, nb)

                drain(c, bstat)
                out_copy(c, bstat).start()

        for t in range(NBUF):
            c = CHUNKS_PER_W - NBUF + t
            out_copy(c, c % NBUF).wait()

    return k(tables_pad, idx_flat)


_BLKC = 10240
ROWS2 = 102400


def _tpad_block(x_ref, o_ref):
    xt = x_ref[0].T
    o_ref[0] = jnp.concatenate(
        [xt, jnp.zeros((_BLKC, DP - D), xt.dtype)], axis=1)


def _tc_transpose_pad(tables_t):
    nj = ROWS2 // _BLKC
    return pl.pallas_call(
        _tpad_block,
        grid=(F, nj),
        in_specs=[pl.BlockSpec((1, D, _BLKC), lambda f, j: (f, 0, j))],
        out_specs=pl.BlockSpec((1, _BLKC, DP), lambda f, j: (f, j, 0)),
        out_shape=jax.ShapeDtypeStruct((F, ROWS2, DP), jnp.float32),
    )(tables_t)


def _mlp_block(x_ref, w1_ref, b1_ref, w2_ref, b2_ref, o_ref):
    x = x_ref[...].astype(jnp.bfloat16)
    w1 = w1_ref[...].astype(jnp.bfloat16)
    l1 = jnp.dot(x, w1, preferred_element_type=jnp.float32) + b1_ref[...]
    l1 = jnp.maximum(l1, 0.0).astype(jnp.bfloat16)
    w2 = w2_ref[...].astype(jnp.bfloat16)
    l2 = jnp.dot(l1, w2, preferred_element_type=jnp.float32) + b2_ref[...]
    o_ref[...] = jax.nn.sigmoid(l2)


def _tc_mlp(x, W1p, b1, W2, b2):
    BLK = 1024
    return pl.pallas_call(
        _mlp_block,
        grid=(B // BLK,),
        in_specs=[
            pl.BlockSpec((BLK, F * DP), lambda i: (i, 0)),
            pl.BlockSpec((F * DP, H), lambda i: (0, 0)),
            pl.BlockSpec((1, H), lambda i: (0, 0)),
            pl.BlockSpec((H, 1), lambda i: (0, 0)),
            pl.BlockSpec((1, 1), lambda i: (0, 0)),
        ],
        out_specs=pl.BlockSpec((BLK, 1), lambda i: (i, 0)),
        out_shape=jax.ShapeDtypeStruct((B, 1), jnp.float32),
    )(x, W1p, b1, W2, b2)


def kernel(xb, tables, W1, b1, W2, b2):
    tables_pad = _tc_transpose_pad(jnp.transpose(tables, (0, 2, 1))).reshape(
        F * ROWS2, DP)
    W1p = jnp.pad(W1.reshape(F, D, H), ((0, 0), (0, DP - D), (0, 0))).reshape(
        F * DP, H)
    idx_flat = (xb + (jnp.arange(F, dtype=jnp.int32) * ROWS2)[None, :]).reshape(
        TOTAL)
    gathered = _sc_gather(tables_pad, idx_flat)
    x = gathered.reshape(B, F * DP)
    return _tc_mlp(x, W1p, b1.reshape(1, H), W2, b2.reshape(1, 1))
